# baseline (speedup 1.0000x reference)
import functools

import jax
import jax.numpy as jnp
from jax.experimental import pallas as pl
from jax.experimental.pallas import tpu as pltpu

_CODE_DIM = 64
_K = 1024
_SOS_TOKEN = 1024
_N_TOK = 65536
_N_ROWS = _N_TOK // 2
_BLOCK_ROWS = 2048
_BLOCK_TOK = 2 * _BLOCK_ROWS


def _riffle(e, o, n):
    c = jnp.concatenate([e, o], axis=1)
    iota = jax.lax.broadcasted_iota(jnp.int32, (1, 2 * n), 1)
    s = n
    while s >= 2:
        q = s // 2
        b = (iota // q) & 3
        cm = pltpu.roll(c, 2 * n - q, 1)
        cp = pltpu.roll(c, q, 1)
        c = jnp.where(b == 1, cm, jnp.where(b == 2, cp, c))
        s = q
    return c


def _vq_argmin_kernel(cb_ref, x_ref, xout_ref, yout_ref,
                      cbe_scr, cbo_scr, carry_scr):
    @pl.when(pl.program_id(0) == 0)
    def _init():
        cbm2 = -2.0 * cb_ref[...]
        cbsq = 0.25 * jnp.sum(cbm2 * cbm2, axis=1, keepdims=True)
        zero = jnp.zeros_like(cbm2)
        lane64 = jax.lax.broadcasted_iota(jnp.int32, (_K, _CODE_DIM), 1)
        eblk = jnp.where(lane64 == 0, cbsq, 0.0)
        oblk = jnp.where(lane64 == 0, cbsq, zero)
        cbe_scr[...] = jnp.concatenate([cbm2, eblk], axis=1)
        cbo_scr[...] = jnp.concatenate(
            [oblk, cbm2], axis=1)
        carry_scr[0] = _SOS_TOKEN

    x2 = x_ref[...]
    lane = jax.lax.broadcasted_iota(jnp.int32, (_BLOCK_ROWS, 2 * _CODE_DIM), 1)
    x2e = jnp.where(lane == _CODE_DIM, 1.0, x2)
    x2o = jnp.where(lane == 0, 1.0, x2)
    dims = (((1,), (1,)), ((), ()))
    de = jax.lax.dot_general(cbe_scr[...], x2e, dims,
                             preferred_element_type=jnp.float32)
    do = jax.lax.dot_general(cbo_scr[...], x2o, dims,
                             preferred_element_type=jnp.float32)
    idx_e = jnp.argmin(de, axis=0).astype(jnp.int32)
    idx_o = jnp.argmin(do, axis=0).astype(jnp.int32)
    y2 = _riffle(idx_e.reshape(1, _BLOCK_ROWS), idx_o.reshape(1, _BLOCK_ROWS),
                 _BLOCK_ROWS)
    yout_ref[...] = y2.reshape(_BLOCK_TOK)
    rolled = pltpu.roll(y2, 1, 1)
    lt = jax.lax.broadcasted_iota(jnp.int32, (1, _BLOCK_TOK), 1)
    xout_ref[...] = jnp.where(lt == 0, carry_scr[0], rolled).reshape(
        _BLOCK_TOK)
    carry_scr[0] = yout_ref[_BLOCK_TOK - 1]


@functools.partial(jax.jit, static_argnames=("interpret",))
def _vq_transform(flat2, codebook, interpret=False):
    return pl.pallas_call(
        _vq_argmin_kernel,
        grid=(_N_ROWS // _BLOCK_ROWS,),
        in_specs=[
            pl.BlockSpec((_K, _CODE_DIM), lambda i: (0, 0)),
            pl.BlockSpec((_BLOCK_ROWS, 2 * _CODE_DIM), lambda i: (i, 0)),
        ],
        out_specs=[
            pl.BlockSpec((_BLOCK_TOK,), lambda i: (i,)),
            pl.BlockSpec((_BLOCK_TOK,), lambda i: (i,)),
        ],
        out_shape=[
            jax.ShapeDtypeStruct((_N_TOK,), jnp.int32),
            jax.ShapeDtypeStruct((_N_TOK,), jnp.int32),
        ],
        scratch_shapes=[
            pltpu.VMEM((_K, 2 * _CODE_DIM), jnp.float32),
            pltpu.VMEM((_K, 2 * _CODE_DIM), jnp.float32),
            pltpu.SMEM((1,), jnp.int32),
        ],
        interpret=interpret,
    )(codebook, flat2)


def kernel(weights_dict, y, codebook):
    flat2 = weights_dict.reshape(_N_ROWS, 2 * _CODE_DIM)
    x_out, y_out = _vq_transform(flat2, codebook)
    return (x_out, y_out)


# speedup vs baseline: 1.1947x; 1.1947x over previous
import functools

import jax
import jax.numpy as jnp
from jax.experimental import pallas as pl
from jax.experimental.pallas import tpu as pltpu

_CODE_DIM = 64
_K = 1024
_SOS_TOKEN = 1024
_N_TOK = 65536
_N_ROWS = _N_TOK // 2
_BLOCK_ROWS = 8192
_BLOCK_TOK = 2 * _BLOCK_ROWS


def _riffle(e, o, n):
    c = jnp.concatenate([e, o], axis=1)
    iota = jax.lax.broadcasted_iota(jnp.int32, (1, 2 * n), 1)
    s = n
    while s >= 2:
        q = s // 2
        b = (iota // q) & 3
        cm = pltpu.roll(c, 2 * n - q, 1)
        cp = pltpu.roll(c, q, 1)
        c = jnp.where(b == 1, cm, jnp.where(b == 2, cp, c))
        s = q
    return c


def _vq_argmin_kernel(cb_ref, x_ref, xout_ref, yout_ref,
                      cbe_scr, cbo_scr, carry_scr):
    @pl.when(pl.program_id(0) == 0)
    def _init():
        cbm2 = -2.0 * cb_ref[...]
        cbsq = 0.25 * jnp.sum(cbm2 * cbm2, axis=1, keepdims=True)
        zero = jnp.zeros_like(cbm2)
        lane64 = jax.lax.broadcasted_iota(jnp.int32, (_K, _CODE_DIM), 1)
        eblk = jnp.where(lane64 == 0, cbsq, 0.0)
        oblk = jnp.where(lane64 == 0, cbsq, zero)
        cbe_scr[...] = jnp.concatenate([cbm2, eblk], axis=1)
        cbo_scr[...] = jnp.concatenate(
            [oblk, cbm2], axis=1)
        carry_scr[0] = _SOS_TOKEN

    x2 = x_ref[...]
    lane = jax.lax.broadcasted_iota(jnp.int32, (_BLOCK_ROWS, 2 * _CODE_DIM), 1)
    x2e = jnp.where(lane == _CODE_DIM, 1.0, x2)
    x2o = jnp.where(lane == 0, 1.0, x2)
    dims = (((1,), (1,)), ((), ()))
    de = jax.lax.dot_general(cbe_scr[...], x2e, dims,
                             preferred_element_type=jnp.float32)
    do = jax.lax.dot_general(cbo_scr[...], x2o, dims,
                             preferred_element_type=jnp.float32)
    idx_e = jnp.argmin(de, axis=0).astype(jnp.int32)
    idx_o = jnp.argmin(do, axis=0).astype(jnp.int32)
    y2 = _riffle(idx_e.reshape(1, _BLOCK_ROWS), idx_o.reshape(1, _BLOCK_ROWS),
                 _BLOCK_ROWS)
    yout_ref[...] = y2.reshape(_BLOCK_TOK)
    rolled = pltpu.roll(y2, 1, 1)
    lt = jax.lax.broadcasted_iota(jnp.int32, (1, _BLOCK_TOK), 1)
    xout_ref[...] = jnp.where(lt == 0, carry_scr[0], rolled).reshape(
        _BLOCK_TOK)
    carry_scr[0] = yout_ref[_BLOCK_TOK - 1]


@functools.partial(jax.jit, static_argnames=("interpret",))
def _vq_transform(flat2, codebook, interpret=False):
    return pl.pallas_call(
        _vq_argmin_kernel,
        grid=(_N_ROWS // _BLOCK_ROWS,),
        in_specs=[
            pl.BlockSpec((_K, _CODE_DIM), lambda i: (0, 0)),
            pl.BlockSpec((_BLOCK_ROWS, 2 * _CODE_DIM), lambda i: (i, 0)),
        ],
        out_specs=[
            pl.BlockSpec((_BLOCK_TOK,), lambda i: (i,)),
            pl.BlockSpec((_BLOCK_TOK,), lambda i: (i,)),
        ],
        out_shape=[
            jax.ShapeDtypeStruct((_N_TOK,), jnp.int32),
            jax.ShapeDtypeStruct((_N_TOK,), jnp.int32),
        ],
        scratch_shapes=[
            pltpu.VMEM((_K, 2 * _CODE_DIM), jnp.float32),
            pltpu.VMEM((_K, 2 * _CODE_DIM), jnp.float32),
            pltpu.SMEM((1,), jnp.int32),
        ],
        interpret=interpret,
    )(codebook, flat2)


def kernel(weights_dict, y, codebook):
    flat2 = weights_dict.reshape(_N_ROWS, 2 * _CODE_DIM)
    x_out, y_out = _vq_transform(flat2, codebook)
    return (x_out, y_out)
